# baseline (speedup 1.0000x reference)
import functools

import jax
import jax.numpy as jnp
from jax import lax
from jax.experimental import pallas as pl
from jax.experimental.pallas import tpu as pltpu
from jax.experimental.pallas import tpu_sc as plsc

_TILE = 1024


def _one_batch(px_ref, gxt_ref, out_ref, k):
    gxt2 = gxt_ref[k]
    n1 = px_ref.shape[1]
    n2 = gxt2.shape[1]
    y2 = 0.25 * jnp.sum(gxt2 * gxt2, axis=0, keepdims=True)

    y2_hi = y2.astype(jnp.bfloat16).astype(jnp.float32)
    y2_lo = y2 - y2_hi
    ones_row = jnp.ones((1, n2), dtype=jnp.float32)
    rhs = jnp.concatenate([gxt2, y2_hi, y2_lo, ones_row, ones_row],
                          axis=0)

    px = px_ref[k]
    x2 = jnp.sum(px * px, axis=1, keepdims=True)
    x2_hi = x2.astype(jnp.bfloat16).astype(jnp.float32)
    x2_lo = x2 - x2_hi
    ones_col = jnp.ones((n1, 1), dtype=jnp.float32)
    lhs = jnp.concatenate([px, ones_col, ones_col, x2_hi, x2_lo],
                          axis=1)

    sum_x = jnp.zeros((1, 1), dtype=jnp.float32)
    min_f = jnp.full((1, n2), jnp.inf, dtype=jnp.bfloat16)
    for i in range(n1 // _TILE):
        lhs_t = lhs[i * _TILE:(i + 1) * _TILE]
        d_t = lax.dot_general(lhs_t, rhs, (((1,), (0,)), ((), ())),
                              preferred_element_type=jnp.float32
                              ).astype(jnp.bfloat16)
        cham_x_t = jnp.maximum(
            jnp.min(d_t, axis=1, keepdims=True).astype(jnp.float32), 0.0)
        sum_x = sum_x + jnp.sum(cham_x_t, axis=(0, 1), keepdims=True)
        min_f = jnp.minimum(min_f, jnp.min(d_t, axis=0, keepdims=True))
    cham_y = jnp.maximum(min_f.astype(jnp.float32), 0.0)
    out_ref[k, :, :] = (sum_x / n1
                        + jnp.sum(cham_y, axis=(0, 1), keepdims=True) / n2)


_BATCHES_PER_STEP = 4


def _chamfer_body(px_ref, gxt_ref, out_ref):
    for k in range(_BATCHES_PER_STEP):
        _one_batch(px_ref, gxt_ref, out_ref, k)


def _tc_chamfer(pred_points, gt_points):
    B, N, D = pred_points.shape
    gt_t = jnp.swapaxes(gt_points, 1, 2) * jnp.float32(-2.0)
    g = _BATCHES_PER_STEP
    per_batch = pl.pallas_call(
        _chamfer_body,
        grid=(B // g,),
        in_specs=[
            pl.BlockSpec((g, N, D), lambda b: (b, 0, 0)),
            pl.BlockSpec((g, D, gt_t.shape[2]), lambda b: (b, 0, 0)),
        ],
        out_specs=pl.BlockSpec((g, 1, 1), lambda b: (b, 0, 0)),
        out_shape=jax.ShapeDtypeStruct((B, 1, 1), jnp.float32),
        compiler_params=pltpu.CompilerParams(
            dimension_semantics=("parallel",)),
    )(pred_points, gt_t)
    return per_batch[:, 0, 0]


_NC, _NS, _L = 2, 16, 16
_NW = _NC * _NS
_SC_B = 4
_CHUNKS = _NW // _SC_B
_N = 2048
_CP = _N // _CHUNKS
_PB = 16


def _sc_body(pred_hbm, gt_hbm, rowp_hbm, miny_hbm,
             pred_v, gt_v, miny_v, rowp_v):
    wid = lax.axis_index("s") * _NC + lax.axis_index("c")
    b = wid // _CHUNKS
    c = wid % _CHUNKS
    pltpu.sync_copy(pred_hbm.at[b, :, pl.ds(c * _CP, _CP)], pred_v)
    pltpu.sync_copy(gt_hbm.at[b], gt_v)

    def init_body(j, carry):
        miny_v[pl.ds(j * _L, _L)] = jnp.full((_L,), jnp.inf, jnp.float32)
        return carry

    lax.fori_loop(0, _N // _L, init_body, 0)

    def iblock(ib, carry):
        base = ib * _PB
        pxv = [pred_v[d, pl.ds(base, _PB)] for d in range(3)]
        pxs = [[pxv[d][t] for d in range(3)] for t in range(_PB)]

        def jbody(j, accs):
            gx = gt_v[0, pl.ds(j * _L, _L)]
            gy = gt_v[1, pl.ds(j * _L, _L)]
            gz = gt_v[2, pl.ds(j * _L, _L)]
            ds_ = []
            new_accs = []
            for t in range(_PB):
                dx = gx - pxs[t][0]
                dy = gy - pxs[t][1]
                dz = gz - pxs[t][2]
                d = dx * dx + dy * dy + dz * dz
                ds_.append(d)
                new_accs.append(jnp.minimum(accs[t], d))
            m = ds_[0]
            for t in range(1, _PB):
                m = jnp.minimum(m, ds_[t])
            sl = pl.ds(j * _L, _L)
            miny_v[sl] = jnp.minimum(miny_v[sl], m)
            return tuple(new_accs)

        accs0 = tuple(jnp.full((_L,), jnp.inf, jnp.float32)
                      for _ in range(_PB))
        accs = lax.fori_loop(0, _N // _L, jbody, accs0)
        for t in range(_PB):
            rowp_v[pl.ds((base + t) * _L, _L)] = accs[t]
        return carry

    lax.fori_loop(0, _CP // _PB, iblock, 0)
    pltpu.sync_copy(rowp_v, rowp_hbm.at[b, c])
    pltpu.sync_copy(miny_v, miny_hbm.at[b, c])


def _sc_chamfer(pred_t, gt_t):
    mesh = plsc.VectorSubcoreMesh(core_axis_name="c", subcore_axis_name="s")
    run = functools.partial(
        pl.kernel, mesh=mesh,
        out_type=[
            jax.ShapeDtypeStruct((_SC_B, _CHUNKS, _CP * _L), jnp.float32),
            jax.ShapeDtypeStruct((_SC_B, _CHUNKS, _N), jnp.float32),
        ],
        scratch_types=[
            pltpu.VMEM((3, _CP), jnp.float32),
            pltpu.VMEM((3, _N), jnp.float32),
            pltpu.VMEM((_N,), jnp.float32),
            pltpu.VMEM((_CP * _L,), jnp.float32),
        ],
    )(_sc_body)
    rowp, miny_p = run(pred_t, gt_t)
    cham_x = jnp.min(rowp.reshape(_SC_B, _CHUNKS, _CP, _L), axis=3)
    sum_x = jnp.sum(cham_x, axis=(1, 2))
    min_y = jnp.min(miny_p, axis=1)
    return sum_x / _N + jnp.sum(min_y, axis=1) / _N


def kernel(pred_points, gt_points):
    B = pred_points.shape[0]
    tc_losses = _tc_chamfer(pred_points[:B - _SC_B], gt_points[:B - _SC_B])
    pred_sc = jnp.swapaxes(pred_points[B - _SC_B:], 1, 2)
    gt_sc = jnp.swapaxes(gt_points[B - _SC_B:], 1, 2)
    sc_losses = _sc_chamfer(pred_sc, gt_sc)
    return (jnp.sum(tc_losses) + jnp.sum(sc_losses)) / B


# speedup vs baseline: 1.8962x; 1.8962x over previous
import functools

import jax
import jax.numpy as jnp
from jax import lax
from jax.experimental import pallas as pl
from jax.experimental.pallas import tpu as pltpu
from jax.experimental.pallas import tpu_sc as plsc

_TILE = 1024


def _one_batch(px_ref, gxt_ref, out_ref, k):
    gxt2 = gxt_ref[k]
    n1 = px_ref.shape[1]
    n2 = gxt2.shape[1]
    y2 = 0.25 * jnp.sum(gxt2 * gxt2, axis=0, keepdims=True)

    y2_hi = y2.astype(jnp.bfloat16).astype(jnp.float32)
    y2_lo = y2 - y2_hi
    ones_row = jnp.ones((1, n2), dtype=jnp.float32)
    rhs = jnp.concatenate([gxt2, y2_hi, y2_lo, ones_row, ones_row],
                          axis=0)

    px = px_ref[k]
    x2 = jnp.sum(px * px, axis=1, keepdims=True)
    x2_hi = x2.astype(jnp.bfloat16).astype(jnp.float32)
    x2_lo = x2 - x2_hi
    ones_col = jnp.ones((n1, 1), dtype=jnp.float32)
    lhs = jnp.concatenate([px, ones_col, ones_col, x2_hi, x2_lo],
                          axis=1)

    sum_x = jnp.zeros((1, 1), dtype=jnp.float32)
    min_f = jnp.full((1, n2), jnp.inf, dtype=jnp.bfloat16)
    for i in range(n1 // _TILE):
        lhs_t = lhs[i * _TILE:(i + 1) * _TILE]
        d_t = lax.dot_general(lhs_t, rhs, (((1,), (0,)), ((), ())),
                              preferred_element_type=jnp.float32
                              ).astype(jnp.bfloat16)
        cham_x_t = jnp.maximum(
            jnp.min(d_t, axis=1, keepdims=True).astype(jnp.float32), 0.0)
        sum_x = sum_x + jnp.sum(cham_x_t, axis=(0, 1), keepdims=True)
        min_f = jnp.minimum(min_f, jnp.min(d_t, axis=0, keepdims=True))
    cham_y = jnp.maximum(min_f.astype(jnp.float32), 0.0)
    out_ref[k, :, :] = (sum_x / n1
                        + jnp.sum(cham_y, axis=(0, 1), keepdims=True) / n2)


_BATCHES_PER_STEP = 4


def _chamfer_body(px_ref, gxt_ref, out_ref):
    for k in range(_BATCHES_PER_STEP):
        _one_batch(px_ref, gxt_ref, out_ref, k)


def _tc_chamfer(pred_points, gt_points):
    B, N, D = pred_points.shape
    gt_t = jnp.swapaxes(gt_points, 1, 2) * jnp.float32(-2.0)
    g = _BATCHES_PER_STEP
    per_batch = pl.pallas_call(
        _chamfer_body,
        grid=(B // g,),
        in_specs=[
            pl.BlockSpec((g, N, D), lambda b: (b, 0, 0)),
            pl.BlockSpec((g, D, gt_t.shape[2]), lambda b: (b, 0, 0)),
        ],
        out_specs=pl.BlockSpec((g, 1, 1), lambda b: (b, 0, 0)),
        out_shape=jax.ShapeDtypeStruct((B, 1, 1), jnp.float32),
        compiler_params=pltpu.CompilerParams(
            dimension_semantics=("parallel",)),
    )(pred_points, gt_t)
    return per_batch[:, 0, 0]


_NC, _NS, _L = 2, 16, 16
_NW = _NC * _NS
_SC_B = 4
_CHUNKS = _NW // _SC_B
_N = 2048
_CP = _N // _CHUNKS
_PB = 16


def _sc_body(pred_hbm, gt_hbm, rowp_hbm, miny_hbm,
             pred_v, gt_v, miny_v, rowp_v):
    wid = lax.axis_index("s") * _NC + lax.axis_index("c")
    b = wid // _CHUNKS
    c = wid % _CHUNKS
    pltpu.sync_copy(pred_hbm.at[b, :, pl.ds(c * _CP, _CP)], pred_v)
    pltpu.sync_copy(gt_hbm.at[b], gt_v)

    def init_body(j, carry):
        miny_v[pl.ds(j * _L, _L)] = jnp.full((_L,), jnp.inf, jnp.float32)
        return carry

    lax.fori_loop(0, _N // _L, init_body, 0)

    def iblock(ib, carry):
        base = ib * _PB
        pxv = [pred_v[d, pl.ds(base, _PB)] for d in range(3)]
        for half in range(2):
            hw = _PB // 2
            pxs = [[pxv[d][half * hw + t] for d in range(3)]
                   for t in range(hw)]

            def jbody(j, accs):
                gx = gt_v[0, pl.ds(j * _L, _L)]
                gy = gt_v[1, pl.ds(j * _L, _L)]
                gz = gt_v[2, pl.ds(j * _L, _L)]
                ds_ = []
                new_accs = []
                for t in range(hw):
                    dx = gx - pxs[t][0]
                    dy = gy - pxs[t][1]
                    dz = gz - pxs[t][2]
                    d = dx * dx + dy * dy + dz * dz
                    ds_.append(d)
                    new_accs.append(jnp.minimum(accs[t], d))
                m = ds_[0]
                for t in range(1, hw):
                    m = jnp.minimum(m, ds_[t])
                sl = pl.ds(j * _L, _L)
                miny_v[sl] = jnp.minimum(miny_v[sl], m)
                return tuple(new_accs)

            accs0 = tuple(jnp.full((_L,), jnp.inf, jnp.float32)
                          for _ in range(hw))
            accs = lax.fori_loop(0, _N // _L, jbody, accs0)
            for t in range(hw):
                rowp_v[pl.ds((base + half * hw + t) * _L, _L)] = accs[t]
        return carry

    lax.fori_loop(0, _CP // _PB, iblock, 0)
    pltpu.sync_copy(rowp_v, rowp_hbm.at[b, c])
    pltpu.sync_copy(miny_v, miny_hbm.at[b, c])


def _sc_chamfer(pred_t, gt_t):
    mesh = plsc.VectorSubcoreMesh(core_axis_name="c", subcore_axis_name="s")
    run = functools.partial(
        pl.kernel, mesh=mesh,
        out_type=[
            jax.ShapeDtypeStruct((_SC_B, _CHUNKS, _CP * _L), jnp.float32),
            jax.ShapeDtypeStruct((_SC_B, _CHUNKS, _N), jnp.float32),
        ],
        scratch_types=[
            pltpu.VMEM((3, _CP), jnp.float32),
            pltpu.VMEM((3, _N), jnp.float32),
            pltpu.VMEM((_N,), jnp.float32),
            pltpu.VMEM((_CP * _L,), jnp.float32),
        ],
    )(_sc_body)
    rowp, miny_p = run(pred_t, gt_t)
    cham_x = jnp.min(rowp.reshape(_SC_B, _CHUNKS, _CP, _L), axis=3)
    sum_x = jnp.sum(cham_x, axis=(1, 2))
    min_y = jnp.min(miny_p, axis=1)
    return sum_x / _N + jnp.sum(min_y, axis=1) / _N


def kernel(pred_points, gt_points):
    B = pred_points.shape[0]
    tc_losses = _tc_chamfer(pred_points[:B - _SC_B], gt_points[:B - _SC_B])
    pred_sc = jnp.swapaxes(pred_points[B - _SC_B:], 1, 2)
    gt_sc = jnp.swapaxes(gt_points[B - _SC_B:], 1, 2)
    sc_losses = _sc_chamfer(pred_sc, gt_sc)
    return (jnp.sum(tc_losses) + jnp.sum(sc_losses)) / B


# speedup vs baseline: 3.7606x; 1.9832x over previous
import jax
import jax.numpy as jnp
from jax import lax
from jax.experimental import pallas as pl
from jax.experimental.pallas import tpu as pltpu

_TILE = 1024


def _one_batch(px_ref, gxt_ref, out_ref, k):
    gxt2 = gxt_ref[k]
    n1 = px_ref.shape[1]
    n2 = gxt2.shape[1]
    y2 = 0.25 * jnp.sum(gxt2 * gxt2, axis=0, keepdims=True)

    y2_hi = y2.astype(jnp.bfloat16).astype(jnp.float32)
    y2_lo = y2 - y2_hi
    ones_row = jnp.ones((1, n2), dtype=jnp.float32)
    rhs = jnp.concatenate([gxt2, y2_hi, y2_lo, ones_row, ones_row],
                          axis=0)

    px = px_ref[k]
    x2 = jnp.sum(px * px, axis=1, keepdims=True)
    x2_hi = x2.astype(jnp.bfloat16).astype(jnp.float32)
    x2_lo = x2 - x2_hi
    ones_col = jnp.ones((n1, 1), dtype=jnp.float32)
    lhs = jnp.concatenate([px, ones_col, ones_col, x2_hi, x2_lo],
                          axis=1)

    sum_x = jnp.zeros((1, 1), dtype=jnp.float32)
    min_f = jnp.full((1, n2), jnp.inf, dtype=jnp.bfloat16)
    for i in range(n1 // _TILE):
        lhs_t = lhs[i * _TILE:(i + 1) * _TILE]
        d_t = lax.dot_general(lhs_t, rhs, (((1,), (0,)), ((), ())),
                              preferred_element_type=jnp.float32).astype(jnp.bfloat16)
        cham_x_t = jnp.maximum(jnp.min(d_t, axis=1, keepdims=True).astype(jnp.float32), 0.0)
        sum_x = sum_x + jnp.sum(cham_x_t, axis=(0, 1), keepdims=True)
        min_f = jnp.minimum(min_f, jnp.min(d_t, axis=0, keepdims=True))
    cham_y = jnp.maximum(min_f.astype(jnp.float32), 0.0)
    out_ref[k, :, :] = (sum_x / n1
                        + jnp.sum(cham_y, axis=(0, 1), keepdims=True) / n2)


_BATCHES_PER_STEP = 4


def _chamfer_body(px_ref, gxt_ref, out_ref):
    for k in range(_BATCHES_PER_STEP):
        _one_batch(px_ref, gxt_ref, out_ref, k)


def kernel(pred_points, gt_points):
    B, N, D = pred_points.shape
    gt_t = jnp.swapaxes(gt_points, 1, 2) * jnp.float32(-2.0)
    g = _BATCHES_PER_STEP
    per_batch = pl.pallas_call(
        _chamfer_body,
        grid=(B // g,),
        in_specs=[
            pl.BlockSpec((g, N, D), lambda b: (b, 0, 0)),
            pl.BlockSpec((g, D, gt_t.shape[2]), lambda b: (b, 0, 0)),
        ],
        out_specs=pl.BlockSpec((g, 1, 1), lambda b: (b, 0, 0)),
        out_shape=jax.ShapeDtypeStruct((B, 1, 1), jnp.float32),
        compiler_params=pltpu.CompilerParams(
            dimension_semantics=("parallel",)),
    )(pred_points, gt_t)
    return jnp.mean(per_batch)
